# baseline (speedup 1.0000x reference)
import functools

import jax
import jax.numpy as jnp
from jax import lax
from jax.experimental import pallas as pl
from jax.experimental.pallas import tpu as pltpu
from jax.experimental.pallas import tpu_sc as plsc

EMB_DIM = 64
PADW = 128
N_ROWS = 1000000
CHUNK_C = 512
NFULL = 1953
TAIL0 = NFULL * CHUNK_C
BATCH = 16384
NC = 2
NS = 16
NW = NC * NS
B_PER_W = BATCH // NW
NGROUP = B_PER_W // 16
KPT = (NFULL + NW - 1) // NW
GATH_ROWS = BATCH + 16


def _extract_body(users_hbm, items_hbm, uft_hbm, vft_hbm, ug_out, vg_out,
                  idxall, hits, buf, stage, bscat, semi0, semi1, sems):
    wid = lax.axis_index("s") * NC + lax.axis_index("c")
    lane = lax.iota(jnp.int32, 16)
    semi = (semi0, semi1)

    def wait_in(p):
        pltpu.make_async_copy(
            uft_hbm.at[:, pl.ds(0, CHUNK_C)], buf.at[p], semi[p]
        ).wait()

    def drain_scat(n):
        for _ in range(n):
            pltpu.make_async_copy(
                uft_hbm.at[pl.ds(0, 16), pl.ds(0, PADW)], stage.at[0], sems
            ).wait()

    for t, (src_idx, src_tab, dst) in enumerate((
        (users_hbm, uft_hbm, ug_out),
        (items_hbm, vft_hbm, vg_out),
    )):
        pltpu.sync_copy(src_idx, idxall)

        def scan_body(vi, cur, src_tab=src_tab):
            x = idxall[pl.ds(vi * 16, 16)]
            mine = ((lax.shift_right_logical(x, 9) & 31) == wid) & (x < TAIL0)
            h = (
                (lax.shift_right_logical(x, 14) << 23)
                + ((x & 511) << 14)
                + (vi * 16 + lane)
            )
            plsc.store_compressed(hits.at[pl.ds(cur, 16)], h, mask=mine)
            cnt = plsc.all_reduce_population_count(mine)
            return cur + cnt[0]

        n_hits = lax.fori_loop(0, BATCH // 16, scan_body, 0)
        hits[pl.ds(n_hits, 16)] = jnp.full((16,), 63 << 23, jnp.int32)
        nv = lax.shift_right_logical(n_hits + 15, 4)

        def start_in(k, p, src_tab=src_tab):
            ci = wid + NW * k

            @pl.when(ci < NFULL)
            def _():
                pltpu.async_copy(
                    src_tab.at[:, pl.ds(ci * CHUNK_C, CHUNK_C)],
                    buf.at[p],
                    semi[p],
                )

        start_in(0, 0)

        def step(k, p, nsc, nv=nv, dst=dst, start_in=start_in):
            start_in(k + 1, 1 - p)

            @pl.when(wid + NW * k < NFULL)
            def _():
                wait_in(p)

            def hv_body(i, nsc, k=k, p=p, dst=dst):
                hv = hits[pl.ds(i * 16, 16)]
                m2 = lax.shift_right_logical(hv, 23) == k
                any_hit = plsc.all_reduce_population_count(m2)[0]

                def do_extract():
                    cvec = lax.shift_right_logical(hv, 14) & 511
                    bvec = hv & 16383
                    sp = lax.rem(nsc, 2)

                    @pl.when(nsc >= 2)
                    def _():
                        drain_scat(1)

                    bsel = sp
                    bscat[bsel, pl.ds(0, 16)] = jnp.where(m2, bvec, BATCH + lane)

                    def dloop(d, _):
                        vals = plsc.load_gather(
                            buf, [jnp.broadcast_to(p, (16,)),
                                  jnp.broadcast_to(d, (16,)), cvec]
                        )
                        plsc.store_scatter(
                            stage,
                            [jnp.broadcast_to(bsel, (16,)), lane,
                             jnp.broadcast_to(d, (16,))],
                            vals,
                        )
                        return 0

                    lax.fori_loop(0, EMB_DIM, dloop, 0)
                    pltpu.async_copy(
                        stage.at[bsel], dst.at[bscat.at[bsel]], sems
                    )

                @pl.when(any_hit > 0)
                def _():
                    do_extract()

                return lax.select(any_hit > 0, nsc + 1, nsc)

            nsc = lax.fori_loop(0, nv, hv_body, nsc)
            return nsc

        def pairbody(jj, nsc, step=step):
            nsc = step(2 * jj, 0, nsc)
            nsc = step(2 * jj + 1, 1, nsc)
            return nsc

        nsc = lax.fori_loop(0, KPT // 2, pairbody, 0)

        @pl.when(nsc >= 1)
        def _():
            drain_scat(1)

        @pl.when(nsc >= 2)
        def _():
            drain_scat(1)


@jax.jit
def _extract(users, items, uft, vft):
    mesh = plsc.VectorSubcoreMesh(core_axis_name="c", subcore_axis_name="s")
    k = functools.partial(
        pl.kernel,
        mesh=mesh,
        compiler_params=pltpu.CompilerParams(needs_layout_passes=False),
        out_type=(
            jax.ShapeDtypeStruct((GATH_ROWS, PADW), jnp.float32),
            jax.ShapeDtypeStruct((GATH_ROWS, PADW), jnp.float32),
        ),
        scratch_types=[
            pltpu.VMEM((BATCH,), jnp.int32),
            pltpu.VMEM((BATCH + 16,), jnp.int32),
            pltpu.VMEM((2, EMB_DIM, CHUNK_C), jnp.float32),
            pltpu.VMEM((2, 16, PADW), jnp.float32),
            pltpu.VMEM((2, 16), jnp.int32),
            pltpu.SemaphoreType.DMA,
            pltpu.SemaphoreType.DMA,
            pltpu.SemaphoreType.DMA,
        ],
    )(_extract_body)
    return k(users, items, uft, vft)


def _dot_body(users_hbm, items_hbm, ug_hbm, vg_hbm, uft_hbm, vft_hbm,
              ub_hbm, ib_hbm, out_hbm,
              idxu, idxv, urows, vrows, bub, bib, tailu, tailv, outv,
              sem_u, sem_v, sem_b):
    wid = lax.axis_index("s") * NC + lax.axis_index("c")
    base = wid * B_PER_W

    pltpu.sync_copy(users_hbm.at[wid], idxu)
    pltpu.sync_copy(items_hbm.at[wid], idxv)
    pltpu.sync_copy(uft_hbm.at[:, pl.ds(TAIL0, EMB_DIM)], tailu)
    pltpu.sync_copy(vft_hbm.at[:, pl.ds(TAIL0, EMB_DIM)], tailv)

    def issue(g, gb):
        pltpu.async_copy(ug_hbm.at[pl.ds(base + g * 16, 16), :], urows.at[gb], sem_u)
        pltpu.async_copy(vg_hbm.at[pl.ds(base + g * 16, 16), :], vrows.at[gb], sem_v)
        pltpu.async_copy(ub_hbm.at[idxu.at[pl.ds(g * 16, 16)]], bub.at[gb], sem_b)
        pltpu.async_copy(ib_hbm.at[idxv.at[pl.ds(g * 16, 16)]], bib.at[gb], sem_b)

    def drain(gb):
        pltpu.make_async_copy(ug_hbm.at[pl.ds(0, 16), :], urows.at[gb], sem_u).wait()
        pltpu.make_async_copy(vg_hbm.at[pl.ds(0, 16), :], vrows.at[gb], sem_v).wait()
        pltpu.make_async_copy(ub_hbm.at[pl.ds(0, 16)], bub.at[gb], sem_b).wait()
        pltpu.make_async_copy(ib_hbm.at[pl.ds(0, 16)], bib.at[gb], sem_b).wait()

    lane = lax.iota(jnp.int32, 16)

    issue(0, 0)

    def gbody(g, _):
        gb = lax.rem(g, 2)

        @pl.when(g + 1 < NGROUP)
        def _():
            issue(g + 1, 1 - gb)

        drain(gb)
        gbv = jnp.broadcast_to(gb, (16,))
        cuv = idxu[pl.ds(g * 16, 16)]
        cvv = idxv[pl.ds(g * 16, 16)]
        mu = cuv >= TAIL0
        mv = cvv >= TAIL0
        tu = jnp.maximum(cuv - TAIL0, 0)
        tv = jnp.maximum(cvv - TAIL0, 0)
        acc0 = bub[gb] + bib[gb]

        def dbody(d, acc):
            dd = jnp.broadcast_to(d, (16,))
            du = plsc.load_gather(urows, [gbv, lane, dd])
            dv = plsc.load_gather(vrows, [gbv, lane, dd])
            du_t = plsc.load_gather(tailu, [dd, tu])
            dv_t = plsc.load_gather(tailv, [dd, tv])
            du = jnp.where(mu, du_t, du)
            dv = jnp.where(mv, dv_t, dv)
            return acc + du * dv

        acc = lax.fori_loop(0, EMB_DIM, dbody, acc0, unroll=8)
        outv[g] = acc
        return 0

    lax.fori_loop(0, NGROUP, gbody, 0)

    pltpu.sync_copy(outv, out_hbm.at[wid])


@jax.jit
def _dot(users_r, items_r, ug, vg, uft, vft, ub, ib):
    mesh = plsc.VectorSubcoreMesh(core_axis_name="c", subcore_axis_name="s")
    k = functools.partial(
        pl.kernel,
        mesh=mesh,
        compiler_params=pltpu.CompilerParams(needs_layout_passes=False),
        out_type=jax.ShapeDtypeStruct((NW, NGROUP, 16), jnp.float32),
        scratch_types=[
            pltpu.VMEM((B_PER_W,), jnp.int32),
            pltpu.VMEM((B_PER_W,), jnp.int32),
            pltpu.VMEM((2, 16, PADW), jnp.float32),
            pltpu.VMEM((2, 16, PADW), jnp.float32),
            pltpu.VMEM((2, 16), jnp.float32),
            pltpu.VMEM((2, 16), jnp.float32),
            pltpu.VMEM((EMB_DIM, EMB_DIM), jnp.float32),
            pltpu.VMEM((EMB_DIM, EMB_DIM), jnp.float32),
            pltpu.VMEM((NGROUP, 16), jnp.float32),
            pltpu.SemaphoreType.DMA,
            pltpu.SemaphoreType.DMA,
            pltpu.SemaphoreType.DMA,
        ],
    )(_dot_body)
    return k(users_r, items_r, ug, vg, uft, vft, ub, ib)


def kernel(users, items, user_factors, item_factors, user_biases, item_biases):
    users_i = users.astype(jnp.int32)
    items_i = items.astype(jnp.int32)
    users_r = users_i.reshape(NW, B_PER_W)
    items_r = items_i.reshape(NW, B_PER_W)
    uft = user_factors.T
    vft = item_factors.T
    ub = user_biases.reshape(-1)
    ib = item_biases.reshape(-1)
    ug, vg = _extract(users_i, items_i, uft, vft)
    out = _dot(users_r, items_r, ug, vg, uft, vft, ub, ib)
    return out.reshape(BATCH)
